# baseline (speedup 1.0000x reference)
import jax
import jax.numpy as jnp
from jax.experimental import pallas as pl
from jax.experimental.pallas import tpu as pltpu

_N = 10000
_D_IN = 128
_D_OUT = 128
_BM = 200


def _gcn_kernel(x_ref, w_ref, b_ref, a_ref, out_ref, temple_ref):
    @pl.when(pl.program_id(0) == 0)
    def _compute_temple():
        temple_ref[...] = jnp.dot(
            x_ref[...], w_ref[...],
            preferred_element_type=jnp.float32,
            precision=jax.lax.Precision.HIGHEST,
        )

    out_ref[...] = (
        jnp.dot(
            a_ref[...], temple_ref[...],
            preferred_element_type=jnp.float32,
            precision=jax.lax.Precision.HIGHEST,
        )
        + b_ref[...]
    )


def kernel(adjacency, features_matrix, weights, bias):
    bias2d = bias.reshape(1, _D_OUT)
    return pl.pallas_call(
        _gcn_kernel,
        grid=(_N // _BM,),
        in_specs=[
            pl.BlockSpec((_N, _D_IN), lambda i: (0, 0)),
            pl.BlockSpec((_D_IN, _D_OUT), lambda i: (0, 0)),
            pl.BlockSpec((1, _D_OUT), lambda i: (0, 0)),
            pl.BlockSpec((_BM, _N), lambda i: (i, 0)),
        ],
        out_specs=pl.BlockSpec((_BM, _D_OUT), lambda i: (i, 0)),
        out_shape=jax.ShapeDtypeStruct((_N, _D_OUT), jnp.float32),
        scratch_shapes=[pltpu.VMEM((_N, _D_IN), jnp.float32)],
    )(features_matrix, weights, bias2d, adjacency)


# speedup vs baseline: 2.6932x; 2.6932x over previous
import jax
import jax.numpy as jnp
from jax.experimental import pallas as pl
from jax.experimental.pallas import tpu as pltpu

_N = 10000
_D_IN = 128
_D_OUT = 128
_BM = 200


def _gcn_kernel(x_ref, w_ref, b_ref, a_ref, out_ref, temple_ref):
    @pl.when(pl.program_id(0) == 0)
    def _compute_temple():
        temple_ref[...] = jnp.dot(
            x_ref[...], w_ref[...],
            preferred_element_type=jnp.float32,
            precision=jax.lax.Precision.HIGHEST,
        )

    out_ref[...] = (
        jnp.dot(
            a_ref[...], temple_ref[...],
            preferred_element_type=jnp.float32,
            precision=jax.lax.Precision.DEFAULT,
        )
        + b_ref[...]
    )


def kernel(adjacency, features_matrix, weights, bias):
    bias2d = bias.reshape(1, _D_OUT)
    return pl.pallas_call(
        _gcn_kernel,
        grid=(_N // _BM,),
        in_specs=[
            pl.BlockSpec((_N, _D_IN), lambda i: (0, 0)),
            pl.BlockSpec((_D_IN, _D_OUT), lambda i: (0, 0)),
            pl.BlockSpec((1, _D_OUT), lambda i: (0, 0)),
            pl.BlockSpec((_BM, _N), lambda i: (i, 0)),
        ],
        out_specs=pl.BlockSpec((_BM, _D_OUT), lambda i: (i, 0)),
        out_shape=jax.ShapeDtypeStruct((_N, _D_OUT), jnp.float32),
        scratch_shapes=[pltpu.VMEM((_N, _D_IN), jnp.float32)],
    )(features_matrix, weights, bias2d, adjacency)
